# baseline (speedup 1.0000x reference)
import functools

import jax
import jax.numpy as jnp
from jax import lax
from jax.experimental import pallas as pl
from jax.experimental.pallas import tpu as pltpu

_NUM_BINS = 29
_OFFSETS_K0 = [(dy, dx) for dy in (-1, 0, 1) for dx in (-1, 0, 1)]
_OFFSETS_K1 = [(dy, dx) for dy in (-3, 0, 3) for dx in (-3, 0, 3)
               if (dy, dx) != (0, 0)]


def _bin_kernel(x_ref, o_ref, *, W):
    N = W * W
    a = x_ref[0]
    DT = a.shape[0]
    dt = a.dtype

    xpos = lax.rem(lax.broadcasted_iota(jnp.int32, (1, N), 1), W)
    ypos = lax.div(lax.broadcasted_iota(jnp.int32, (1, N), 1), W)
    m_lo = xpos == 0
    m_hi = xpos == W - 1

    def rotl(v):
        return jnp.concatenate([v[:, 1:], v[:, :1]], axis=1)

    def rotr(v):
        return jnp.concatenate([v[:, N - 1:], v[:, :N - 1]], axis=1)

    def col_shift(v, dx):
        for _ in range(abs(dx)):
            if dx > 0:
                v = jnp.where(m_hi, v, rotl(v))
            else:
                v = jnp.where(m_lo, v, rotr(v))
        return v

    def row_shift(v, dy):
        s = dy * W
        if dy > 0:
            return jnp.concatenate([v[:, s:]] + [v[:, N - W:]] * dy, axis=1)
        if dy < 0:
            return jnp.concatenate([v[:, :W]] * (-dy) + [v[:, :N + s]], axis=1)
        return v

    csum = a + jnp.where(m_hi, 0.0, rotl(a)) + jnp.where(m_lo, 0.0, rotr(a))
    zrow = jnp.zeros((DT, W), dt)
    num = (csum
           + jnp.concatenate([csum[:, W:], zrow], axis=1)
           + jnp.concatenate([zrow, csum[:, :N - W]], axis=1))
    cx = jnp.where(m_lo | m_hi, 2.0, 3.0)
    cy = jnp.where((ypos == 0) | (ypos == W - 1), 2.0, 3.0)
    pool = num * (1.0 / (cx * cy))

    cols_a = {dx: col_shift(a, dx) for dx in (-1, 0, 1)}
    cols_p = {dx: col_shift(pool, dx) for dx in (-3, 0, 3)}

    j = 0
    for dy, dx in _OFFSETS_K0:
        o_ref[0, j] = row_shift(cols_a[dx], dy)
        j += 1
    for dy, dx in _OFFSETS_K1:
        o_ref[0, j] = row_shift(cols_p[dx], dy)
        j += 1
    o_ref[0, j:] = jnp.zeros((_NUM_BINS - j, DT, N), dt)


def kernel(x):
    B, w0, h0, d = x.shape
    N = w0 * h0
    DT = 32
    xt = jnp.transpose(x, (0, 3, 1, 2)).reshape(B, d, N)
    out = pl.pallas_call(
        functools.partial(_bin_kernel, W=w0),
        out_shape=jax.ShapeDtypeStruct((B, _NUM_BINS, d, N), x.dtype),
        grid=(B, d // DT),
        in_specs=[pl.BlockSpec((1, DT, N), lambda b, i: (b, i, 0))],
        out_specs=pl.BlockSpec((1, _NUM_BINS, DT, N), lambda b, i: (b, 0, i, 0)),
        compiler_params=pltpu.CompilerParams(
            dimension_semantics=("parallel", "parallel"),
        ),
        name="logbin_descriptor",
    )(xt)
    return out.reshape(B, _NUM_BINS * d, w0, h0)


# speedup vs baseline: 1.0587x; 1.0587x over previous
import functools

import jax
import jax.numpy as jnp
from jax import lax
from jax.experimental import pallas as pl
from jax.experimental.pallas import tpu as pltpu

_NUM_BINS = 29
_OFFSETS_K0 = [(dy, dx) for dy in (-1, 0, 1) for dx in (-1, 0, 1)]
_OFFSETS_K1 = [(dy, dx) for dy in (-3, 0, 3) for dx in (-3, 0, 3)
               if (dy, dx) != (0, 0)]


def _bin_kernel(x_ref, o_ref, *, W):
    a = x_ref[0]
    DT = a.shape[0]
    dt = a.dtype

    def shift(v, d, axis, clamp):
        if d == 0:
            return v
        idx = [slice(None)] * 3
        edge = [slice(None)] * 3
        if d > 0:
            idx[axis] = slice(d, W)
            edge[axis] = slice(W - 1, W)
        else:
            idx[axis] = slice(0, W + d)
            edge[axis] = slice(0, 1)
        body = v[tuple(idx)]
        if clamp:
            pad = [v[tuple(edge)]] * abs(d)
        else:
            zshape = list(v.shape)
            zshape[axis] = abs(d)
            pad = [jnp.zeros(zshape, dt)]
        parts = [body] + pad if d > 0 else pad + [body]
        return jnp.concatenate(parts, axis=axis)

    xs = a + shift(a, 1, 2, False) + shift(a, -1, 2, False)
    num = xs + shift(xs, 1, 1, False) + shift(xs, -1, 1, False)
    iy = lax.broadcasted_iota(jnp.int32, (1, W, W), 1)
    ix = lax.broadcasted_iota(jnp.int32, (1, W, W), 2)
    cy = jnp.where((iy == 0) | (iy == W - 1), 2.0, 3.0)
    cx = jnp.where((ix == 0) | (ix == W - 1), 2.0, 3.0)
    pool = num * (1.0 / (cx * cy))

    j = 0
    for dy, dx in _OFFSETS_K0:
        o_ref[0, j] = shift(shift(a, dx, 2, True), dy, 1, True)
        j += 1
    for dy, dx in _OFFSETS_K1:
        o_ref[0, j] = shift(shift(pool, dx, 2, True), dy, 1, True)
        j += 1
    o_ref[0, j:] = jnp.zeros((_NUM_BINS - j, DT, W, W), dt)


def kernel(x):
    B, w0, h0, d = x.shape
    DT = 32
    xt = jnp.transpose(x, (0, 3, 1, 2))
    out = pl.pallas_call(
        functools.partial(_bin_kernel, W=w0),
        out_shape=jax.ShapeDtypeStruct((B, _NUM_BINS, d, w0, h0), x.dtype),
        grid=(B, d // DT),
        in_specs=[pl.BlockSpec((1, DT, w0, h0), lambda b, i: (b, i, 0, 0))],
        out_specs=pl.BlockSpec((1, _NUM_BINS, DT, w0, h0),
                               lambda b, i: (b, 0, i, 0, 0)),
        compiler_params=pltpu.CompilerParams(
            dimension_semantics=("parallel", "arbitrary"),
            vmem_limit_bytes=52 * 1024 * 1024,
        ),
        name="logbin_descriptor",
    )(xt)
    return out.reshape(B, _NUM_BINS * d, w0, h0)
